# baseline (speedup 1.0000x reference)
import jax
import jax.numpy as jnp
from jax.experimental import pallas as pl
from jax.experimental.pallas import tpu as pltpu

_NUM_LIGHTING = 16
_NUM_SAMPLE = 262144
_NUM_CHANNEL = 3
_LANES = 128
_SUBLANES = 8
_TROW = _NUM_LIGHTING // _SUBLANES
_TCOL = _NUM_SAMPLE // _LANES
_NSPLIT = 8
_JBLK = _TCOL // _NSPLIT


def _copy_row_body(idx_ref, tab, out, buf, rsems, wsems):
    row = idx_ref[...]
    trow = row // _SUBLANES
    sub = row % _SUBLANES
    for i in range(_NSPLIT):
        j0 = i * _JBLK
        for c in range(_NUM_CHANNEL):
            pltpu.make_async_copy(
                tab.at[c, trow, pl.ds(j0, _JBLK), sub, :],
                buf.at[c, pl.ds(j0, _JBLK), :],
                rsems.at[i * _NUM_CHANNEL + c],
            ).start()
    for i in range(_NSPLIT):
        j0 = i * _JBLK
        for c in range(_NUM_CHANNEL):
            pltpu.make_async_copy(
                tab.at[c, trow, pl.ds(j0, _JBLK), sub, :],
                buf.at[c, pl.ds(j0, _JBLK), :],
                rsems.at[i * _NUM_CHANNEL + c],
            ).wait()
            pltpu.make_async_copy(
                buf.at[c, pl.ds(j0, _JBLK), :],
                out.at[c, 0, pl.ds(j0, _JBLK), :],
                wsems.at[i * _NUM_CHANNEL + c],
            ).start()
    for i in range(_NSPLIT):
        j0 = i * _JBLK
        for c in range(_NUM_CHANNEL):
            pltpu.make_async_copy(
                buf.at[c, pl.ds(j0, _JBLK), :],
                out.at[c, 0, pl.ds(j0, _JBLK), :],
                wsems.at[i * _NUM_CHANNEL + c],
            ).wait()


def kernel(l_samples, l_dir, lighting_idx):
    del l_dir
    tab = (
        l_samples.transpose(2, 0, 1)
        .reshape(_NUM_CHANNEL, _TROW, _SUBLANES, _TCOL, _LANES)
        .transpose(0, 1, 3, 2, 4)
    )
    idx1 = jnp.asarray(lighting_idx, jnp.int32)

    out = pl.pallas_call(
        _copy_row_body,
        grid_spec=pltpu.PrefetchScalarGridSpec(
            num_scalar_prefetch=1,
            grid=(),
            in_specs=[pl.BlockSpec(memory_space=pl.ANY)],
            out_specs=pl.BlockSpec(memory_space=pl.ANY),
            scratch_shapes=[
                pltpu.VMEM((_NUM_CHANNEL, _TCOL, _LANES), jnp.float32),
                pltpu.SemaphoreType.DMA((_NSPLIT * _NUM_CHANNEL,)),
                pltpu.SemaphoreType.DMA((_NSPLIT * _NUM_CHANNEL,)),
            ],
        ),
        out_shape=jax.ShapeDtypeStruct(
            (_NUM_CHANNEL, 1, _TCOL, _LANES), jnp.float32
        ),
    )(idx1, tab)
    return out.reshape(_NUM_CHANNEL, 1, _NUM_SAMPLE).transpose(1, 2, 0)


# speedup vs baseline: 1.0925x; 1.0925x over previous
import jax
import jax.numpy as jnp
from jax.experimental import pallas as pl
from jax.experimental.pallas import tpu as pltpu

_NUM_LIGHTING = 16
_NUM_SAMPLE = 262144
_NUM_CHANNEL = 3
_LANES = 128
_SUBLANES = 8
_TROW = _NUM_LIGHTING // _SUBLANES
_TCOL = _NUM_SAMPLE // _LANES
_SPLITS = (512, 384, 384, 256, 192, 128, 128, 64)
assert sum(_SPLITS) == _TCOL
_NSPLIT = len(_SPLITS)
_OFFS = tuple(sum(_SPLITS[:i]) for i in range(_NSPLIT))


def _copy_row_body(idx_ref, tab, out, buf, rsems, wsems):
    row = idx_ref[0]
    trow = row // _SUBLANES
    sub = row % _SUBLANES
    for i in range(_NSPLIT):
        j0, jn = _OFFS[i], _SPLITS[i]
        pltpu.make_async_copy(
            tab.at[:, trow, pl.ds(j0, jn), sub, :],
            buf.at[:, pl.ds(j0, jn), :],
            rsems.at[i],
        ).start()
    for i in range(_NSPLIT):
        j0, jn = _OFFS[i], _SPLITS[i]
        pltpu.make_async_copy(
            tab.at[:, trow, pl.ds(j0, jn), sub, :],
            buf.at[:, pl.ds(j0, jn), :],
            rsems.at[i],
        ).wait()
        pltpu.make_async_copy(
            buf.at[:, pl.ds(j0, jn), :],
            out.at[:, 0, pl.ds(j0, jn), :],
            wsems.at[i],
        ).start()
    for i in range(_NSPLIT):
        j0, jn = _OFFS[i], _SPLITS[i]
        pltpu.make_async_copy(
            buf.at[:, pl.ds(j0, jn), :],
            out.at[:, 0, pl.ds(j0, jn), :],
            wsems.at[i],
        ).wait()


def kernel(l_samples, l_dir, lighting_idx):
    del l_dir
    tab = (
        l_samples.transpose(2, 0, 1)
        .reshape(_NUM_CHANNEL, _TROW, _SUBLANES, _TCOL, _LANES)
        .transpose(0, 1, 3, 2, 4)
    )
    idx1 = jnp.asarray(lighting_idx, jnp.int32).reshape(1)

    out = pl.pallas_call(
        _copy_row_body,
        grid_spec=pltpu.PrefetchScalarGridSpec(
            num_scalar_prefetch=1,
            grid=(),
            in_specs=[pl.BlockSpec(memory_space=pl.ANY)],
            out_specs=pl.BlockSpec(memory_space=pl.ANY),
            scratch_shapes=[
                pltpu.VMEM((_NUM_CHANNEL, _TCOL, _LANES), jnp.float32),
                pltpu.SemaphoreType.DMA((_NSPLIT,)),
                pltpu.SemaphoreType.DMA((_NSPLIT,)),
            ],
        ),
        out_shape=jax.ShapeDtypeStruct(
            (_NUM_CHANNEL, 1, _TCOL, _LANES), jnp.float32
        ),
    )(idx1, tab)
    return out.reshape(_NUM_CHANNEL, 1, _NUM_SAMPLE).transpose(1, 2, 0)


# speedup vs baseline: 1.1097x; 1.0157x over previous
import jax
import jax.numpy as jnp
from jax.experimental import pallas as pl
from jax.experimental.pallas import tpu as pltpu

_NUM_LIGHTING = 16
_NUM_SAMPLE = 262144
_NUM_CHANNEL = 3
_LANES = 128
_SUBLANES = 8
_TROW = _NUM_LIGHTING // _SUBLANES
_TCOL = _NUM_SAMPLE // _LANES
_NSPLIT = 8
_JBLK = _TCOL // _NSPLIT


def _copy_row_body(idx_ref, tab, out, buf, rsems, wsems):
    row = idx_ref[0]
    trow = row // _SUBLANES
    sub = row % _SUBLANES
    for i in range(_NSPLIT):
        j0 = i * _JBLK
        pltpu.make_async_copy(
            tab.at[:, trow, pl.ds(j0, _JBLK), sub, :],
            buf.at[:, pl.ds(j0, _JBLK), :],
            rsems.at[i],
        ).start()
    for i in range(_NSPLIT):
        j0 = i * _JBLK
        pltpu.make_async_copy(
            tab.at[:, trow, pl.ds(j0, _JBLK), sub, :],
            buf.at[:, pl.ds(j0, _JBLK), :],
            rsems.at[i],
        ).wait()
        pltpu.make_async_copy(
            buf.at[:, pl.ds(j0, _JBLK), :],
            out.at[:, 0, pl.ds(j0, _JBLK), :],
            wsems.at[i],
        ).start()
    for i in range(_NSPLIT):
        j0 = i * _JBLK
        pltpu.make_async_copy(
            buf.at[:, pl.ds(j0, _JBLK), :],
            out.at[:, 0, pl.ds(j0, _JBLK), :],
            wsems.at[i],
        ).wait()


def kernel(l_samples, l_dir, lighting_idx):
    del l_dir
    tab = (
        l_samples.transpose(2, 0, 1)
        .reshape(_NUM_CHANNEL, _TROW, _SUBLANES, _TCOL, _LANES)
        .transpose(0, 1, 3, 2, 4)
    )
    idx1 = jnp.asarray(lighting_idx, jnp.int32).reshape(1)

    out = pl.pallas_call(
        _copy_row_body,
        grid_spec=pltpu.PrefetchScalarGridSpec(
            num_scalar_prefetch=1,
            grid=(),
            in_specs=[pl.BlockSpec(memory_space=pl.ANY)],
            out_specs=pl.BlockSpec(memory_space=pl.ANY),
            scratch_shapes=[
                pltpu.VMEM((_NUM_CHANNEL, _TCOL, _LANES), jnp.float32),
                pltpu.SemaphoreType.DMA((_NSPLIT,)),
                pltpu.SemaphoreType.DMA((_NSPLIT,)),
            ],
        ),
        out_shape=jax.ShapeDtypeStruct(
            (_NUM_CHANNEL, 1, _TCOL, _LANES), jnp.float32
        ),
    )(idx1, tab)
    return out.reshape(_NUM_CHANNEL, 1, _NUM_SAMPLE).transpose(1, 2, 0)
